# baseline (speedup 1.0000x reference)
import math

import jax
import jax.numpy as jnp
from jax.experimental import pallas as pl
from jax.experimental.pallas import tpu as pltpu

_W = 128


def _body(x_ref, pos_ref, batch_ref, wtp_ref, wnsc_ref, xv_ref, wvsc_ref,
          wn2v_ref, xvo_ref, out_ref, seg_ref, y2_ref):
    ph = pl.program_id(0)
    i = pl.program_id(1)
    d = x_ref.shape[1]
    g = seg_ref.shape[0]
    bb = batch_ref[0]
    b = bb.shape[1]
    lo = batch_ref[0, 0, 0]
    hi = batch_ref[0, 0, b - 1]
    g0 = jnp.minimum((lo // 8) * 8, g - _W)
    fits = (hi - g0) < _W

    @pl.when(ph == 0)
    def _accumulate():
        @pl.when(i == 0)
        def _init():
            seg_ref[...] = jnp.zeros_like(seg_ref)

        x = x_ref[...]
        pos = pos_ref[...]
        p = pos.shape[1]
        n_over_g = pl.num_programs(1) * b / g
        scale = 1.0 / (math.sqrt(d * p) * math.sqrt(n_over_g))
        m = None
        for j in range(p):
            zj = jnp.dot(x, wtp_ref[:, j * d:(j + 1) * d],
                         preferred_element_type=jnp.float32)
            mj = pos[:, j:j + 1] * zj
            m = mj if m is None else m + mj
        m = m * scale

        @pl.when(fits)
        def _narrow():
            onehot_t = (jax.lax.broadcasted_iota(jnp.int32, (_W, b), 0) + g0
                        == bb).astype(jnp.float32)
            seg_ref[pl.ds(g0, _W), :] += jnp.dot(
                onehot_t, m, preferred_element_type=jnp.float32)

        @pl.when(jnp.logical_not(fits))
        def _wide():
            for k in range(g // _W):
                onehot_t = (jax.lax.broadcasted_iota(jnp.int32, (_W, b), 0)
                            + k * _W == bb).astype(jnp.float32)
                seg_ref[k * _W:(k + 1) * _W, :] += jnp.dot(
                    onehot_t, m, preferred_element_type=jnp.float32)

    @pl.when((ph == 1) & (i == 0))
    def _combine():
        sv = jnp.dot(xv_ref[...], wvsc_ref[...],
                     preferred_element_type=jnp.float32) * (1.0 / math.sqrt(d))
        mv = seg_ref[...]
        mv = mv * jax.nn.sigmoid(mv)
        xvo = (sv + mv) * (1.0 / math.sqrt(2.0))
        xvo_ref[...] = xvo
        y2_ref[...] = jnp.dot(xvo, wn2v_ref[...],
                              preferred_element_type=jnp.float32) * (1.0 / math.sqrt(d))

    @pl.when(ph == 1)
    def _node_out():
        x = x_ref[...]
        s = jnp.dot(x, wnsc_ref[...],
                    preferred_element_type=jnp.float32) * (1.0 / math.sqrt(d))

        def _finish(gath):
            out_ref[...] = (s + gath * jax.nn.sigmoid(gath)) * 0.5

        @pl.when(fits)
        def _narrow():
            onehot_t = (jax.lax.broadcasted_iota(jnp.int32, (_W, b), 0) + g0
                        == bb).astype(jnp.float32)
            _finish(jax.lax.dot_general(
                onehot_t, y2_ref[pl.ds(g0, _W), :], (((0,), (0,)), ((), ())),
                preferred_element_type=jnp.float32))

        @pl.when(jnp.logical_not(fits))
        def _wide():
            gath = None
            for k in range(g // _W):
                onehot_t = (jax.lax.broadcasted_iota(jnp.int32, (_W, b), 0)
                            + k * _W == bb).astype(jnp.float32)
                gk = jax.lax.dot_general(
                    onehot_t, y2_ref[k * _W:(k + 1) * _W, :],
                    (((0,), (0,)), ((), ())),
                    preferred_element_type=jnp.float32)
                gath = gk if gath is None else gath + gk
            _finish(gath)


def kernel(x_virtual, x_node, node_pos_sh, batch, W_vsc, W_nsc, W_tp, W_n2v):
    n, d = x_node.shape
    p = node_pos_sh.shape[1]
    g = x_virtual.shape[0]

    B = 10000
    nb = n // B
    assert nb * B == n

    wtp_flat = W_tp.reshape(d, p * d)
    batch3d = batch.reshape(nb, 1, B)

    xvo, x_node_out = pl.pallas_call(
        _body,
        grid=(2, nb),
        in_specs=[
            pl.BlockSpec((B, d), lambda ph, i: (i, 0)),
            pl.BlockSpec((B, p), lambda ph, i: (i, 0)),
            pl.BlockSpec((1, 1, B), lambda ph, i: (i, 0, 0)),
            pl.BlockSpec((d, p * d), lambda ph, i: (0, 0)),
            pl.BlockSpec((d, d), lambda ph, i: (0, 0)),
            pl.BlockSpec((g, d), lambda ph, i: (0, 0)),
            pl.BlockSpec((d, d), lambda ph, i: (0, 0)),
            pl.BlockSpec((d, d), lambda ph, i: (0, 0)),
        ],
        out_specs=(
            pl.BlockSpec((g, d), lambda ph, i: (0, 0)),
            pl.BlockSpec((B, d), lambda ph, i: (ph * i, 0)),
        ),
        out_shape=(jax.ShapeDtypeStruct((g, d), jnp.float32),
                   jax.ShapeDtypeStruct((n, d), jnp.float32)),
        scratch_shapes=[
            pltpu.VMEM((g, d), jnp.float32),
            pltpu.VMEM((g, d), jnp.float32),
        ],
        compiler_params=pltpu.CompilerParams(
            dimension_semantics=("arbitrary", "arbitrary")),
    )(x_node, node_pos_sh, batch3d, wtp_flat, W_nsc, x_virtual, W_vsc, W_n2v)

    return (xvo, x_node_out)


# speedup vs baseline: 1.0554x; 1.0554x over previous
import math

import jax
import jax.numpy as jnp
from jax.experimental import pallas as pl
from jax.experimental.pallas import tpu as pltpu

_W = 128


def _stage_a_body(x_ref, pos_ref, batch_ref, wtp_ref, seg_ref):
    i = pl.program_id(0)

    @pl.when(i == 0)
    def _init():
        seg_ref[...] = jnp.zeros_like(seg_ref)

    x = x_ref[...]
    pos = pos_ref[...]
    d = x.shape[1]
    p = pos.shape[1]
    n_over_g = pl.num_programs(0) * x.shape[0] / seg_ref.shape[0]
    scale = 1.0 / (math.sqrt(d * p) * math.sqrt(n_over_g))
    m = None
    for j in range(p):
        zj = jnp.dot(x, wtp_ref[:, j * d:(j + 1) * d],
                     preferred_element_type=jnp.float32)
        mj = pos[:, j:j + 1] * zj
        m = mj if m is None else m + mj
    m = m * scale
    bb = batch_ref[0]
    b = bb.shape[1]
    g = seg_ref.shape[0]
    lo = batch_ref[0, 0, 0]
    hi = batch_ref[0, 0, b - 1]
    g0 = jnp.minimum((lo // 8) * 8, g - _W)
    fits = (hi - g0) < _W

    @pl.when(fits)
    def _narrow():
        onehot_t = (jax.lax.broadcasted_iota(jnp.int32, (_W, b), 0) + g0
                    == bb).astype(jnp.float32)
        seg_ref[pl.ds(g0, _W), :] += jnp.dot(
            onehot_t, m, preferred_element_type=jnp.float32)

    @pl.when(jnp.logical_not(fits))
    def _wide():
        for k in range(g // _W):
            onehot_t = (jax.lax.broadcasted_iota(jnp.int32, (_W, b), 0)
                        + k * _W == bb).astype(jnp.float32)
            seg_ref[k * _W:(k + 1) * _W, :] += jnp.dot(
                onehot_t, m, preferred_element_type=jnp.float32)


def _stage_c_body(x_ref, batch_ref, wnsc_ref, xv_ref, wvsc_ref, wn2v_ref,
                  seg_ref, xvo_ref, out_ref, y2_ref):
    i = pl.program_id(0)
    d = x_ref.shape[1]

    @pl.when(i == 0)
    def _combine():
        sv = jnp.dot(xv_ref[...], wvsc_ref[...],
                     preferred_element_type=jnp.float32) * (1.0 / math.sqrt(d))
        mv = seg_ref[...]
        mv = mv * jax.nn.sigmoid(mv)
        xvo = (sv + mv) * (1.0 / math.sqrt(2.0))
        xvo_ref[...] = xvo
        y2_ref[...] = jnp.dot(xvo, wn2v_ref[...],
                              preferred_element_type=jnp.float32) * (1.0 / math.sqrt(d))

    x = x_ref[...]
    s = jnp.dot(x, wnsc_ref[...],
                preferred_element_type=jnp.float32) * (1.0 / math.sqrt(d))
    bb = batch_ref[0]
    b = bb.shape[1]
    g = y2_ref.shape[0]
    lo = batch_ref[0, 0, 0]
    hi = batch_ref[0, 0, b - 1]
    g0 = jnp.minimum((lo // 8) * 8, g - _W)
    fits = (hi - g0) < _W

    def _finish(gath):
        out_ref[...] = (s + gath * jax.nn.sigmoid(gath)) * 0.5

    @pl.when(fits)
    def _narrow():
        onehot_t = (jax.lax.broadcasted_iota(jnp.int32, (_W, b), 0) + g0
                    == bb).astype(jnp.float32)
        _finish(jax.lax.dot_general(
            onehot_t, y2_ref[pl.ds(g0, _W), :], (((0,), (0,)), ((), ())),
            preferred_element_type=jnp.float32))

    @pl.when(jnp.logical_not(fits))
    def _wide():
        gath = None
        for k in range(g // _W):
            onehot_t = (jax.lax.broadcasted_iota(jnp.int32, (_W, b), 0)
                        + k * _W == bb).astype(jnp.float32)
            gk = jax.lax.dot_general(
                onehot_t, y2_ref[k * _W:(k + 1) * _W, :],
                (((0,), (0,)), ((), ())),
                preferred_element_type=jnp.float32)
            gath = gk if gath is None else gath + gk
        _finish(gath)


def kernel(x_virtual, x_node, node_pos_sh, batch, W_vsc, W_nsc, W_tp, W_n2v):
    n, d = x_node.shape
    p = node_pos_sh.shape[1]
    g = x_virtual.shape[0]

    B = 10000
    nb = n // B
    assert nb * B == n

    wtp_flat = W_tp.reshape(d, p * d)
    batch3d = batch.reshape(nb, 1, B)

    seg = pl.pallas_call(
        _stage_a_body,
        grid=(nb,),
        in_specs=[
            pl.BlockSpec((B, d), lambda i: (i, 0)),
            pl.BlockSpec((B, p), lambda i: (i, 0)),
            pl.BlockSpec((1, 1, B), lambda i: (i, 0, 0)),
            pl.BlockSpec((d, p * d), lambda i: (0, 0)),
        ],
        out_specs=pl.BlockSpec((g, d), lambda i: (0, 0)),
        out_shape=jax.ShapeDtypeStruct((g, d), jnp.float32),
        compiler_params=pltpu.CompilerParams(
            dimension_semantics=("arbitrary",)),
    )(x_node, node_pos_sh, batch3d, wtp_flat)

    xvo, x_node_out = pl.pallas_call(
        _stage_c_body,
        grid=(nb,),
        in_specs=[
            pl.BlockSpec((B, d), lambda i: (i, 0)),
            pl.BlockSpec((1, 1, B), lambda i: (i, 0, 0)),
            pl.BlockSpec((d, d), lambda i: (0, 0)),
            pl.BlockSpec((g, d), lambda i: (0, 0)),
            pl.BlockSpec((d, d), lambda i: (0, 0)),
            pl.BlockSpec((d, d), lambda i: (0, 0)),
            pl.BlockSpec((g, d), lambda i: (0, 0)),
        ],
        out_specs=(pl.BlockSpec((g, d), lambda i: (0, 0)),
                   pl.BlockSpec((B, d), lambda i: (i, 0))),
        out_shape=(jax.ShapeDtypeStruct((g, d), jnp.float32),
                   jax.ShapeDtypeStruct((n, d), jnp.float32)),
        scratch_shapes=[pltpu.VMEM((g, d), jnp.float32)],
        compiler_params=pltpu.CompilerParams(
            dimension_semantics=("arbitrary",)),
    )(x_node, batch3d, W_nsc, x_virtual, W_vsc, W_n2v, seg)

    return (xvo, x_node_out)
